# Initial kernel scaffold; baseline (speedup 1.0000x reference)
import jax
import jax.numpy as jnp
from jax.experimental import pallas as pl


def kernel(x, pos, edge_index, edge_attr, batch, params):
    raise NotImplementedError("write your pallas kernel here")


# speedup vs baseline: 1.7207x; 1.7207x over previous
import functools

import jax
import jax.numpy as jnp
from jax import lax
from jax.experimental import pallas as pl
from jax.experimental.pallas import tpu as pltpu
from jax.experimental.pallas import tpu_sc as plsc

N = 10000
E = 320000
H = 128
RBF = 16
G = 64
W_TAB = 144

NC = 2
NS = 16
NW = NC * NS
EPW = E // NW
CH = 80
NCHUNK = EPW // CH
ROWS_PER_TILE = N // NS

NB = 1000
EB = 1280
GRID_N = N // NB
GRID_E = E // EB

_mesh = plsc.VectorSubcoreMesh(
    core_axis_name="c", subcore_axis_name="s", num_cores=NC, num_subcores=NS)


@functools.partial(
    pl.kernel,
    mesh=_mesh,
    out_type=[jax.ShapeDtypeStruct((E, W_TAB), jnp.float32),
              jax.ShapeDtypeStruct((E, W_TAB), jnp.float32)],
    scratch_types=[
        pltpu.VMEM((CH,), jnp.int32),
        pltpu.VMEM((CH,), jnp.int32),
        pltpu.VMEM((CH, W_TAB), jnp.float32),
        pltpu.VMEM((CH, W_TAB), jnp.float32),
        pltpu.SemaphoreType.DMA,
        pltpu.SemaphoreType.DMA,
    ],
    compiler_params=pltpu.CompilerParams(use_tc_tiling_on_sc=False),
)
def _sc_gather(a_hbm, b_hbm, row_hbm, col_hbm, sa_hbm, sb_hbm,
               rowv, colv, bufa, bufb, sema, semb):
    wid = lax.axis_index("s") * NC + lax.axis_index("c")
    base = wid * EPW

    def step(k, _):
        off = base + k * CH
        pltpu.sync_copy(row_hbm.at[pl.ds(off, CH)], rowv)
        pltpu.sync_copy(col_hbm.at[pl.ds(off, CH)], colv)
        ca = pltpu.async_copy(a_hbm.at[rowv], bufa, sema)
        cb = pltpu.async_copy(b_hbm.at[colv], bufb, semb)
        ca.wait()
        cb.wait()
        pltpu.sync_copy(bufa, sa_hbm.at[pl.ds(off, CH)])
        pltpu.sync_copy(bufb, sb_hbm.at[pl.ds(off, CH)])
        return 0

    lax.fori_loop(0, NCHUNK, step, 0)


@functools.partial(
    pl.kernel,
    mesh=_mesh,
    out_type=jax.ShapeDtypeStruct((NC, N, W_TAB), jnp.float32),
    scratch_types=[
        pltpu.VMEM((CH,), jnp.int32),
        pltpu.VMEM((CH, W_TAB), jnp.float32),
        pltpu.VMEM_SHARED((N, W_TAB), jnp.float32),
    ],
    compiler_params=pltpu.CompilerParams(use_tc_tiling_on_sc=False),
)
def _sc_scatter(p_hbm, row_hbm, zero_hbm, q_hbm, rowv, bufp, acc):
    cid = lax.axis_index("c")
    sid = lax.axis_index("s")
    wid = sid * NC + cid
    base = wid * EPW
    rbase = sid * ROWS_PER_TILE

    pltpu.sync_copy(zero_hbm.at[pl.ds(rbase, ROWS_PER_TILE)],
                    acc.at[pl.ds(rbase, ROWS_PER_TILE)])
    plsc.subcore_barrier()

    def step(k, _):
        off = base + k * CH
        pltpu.sync_copy(row_hbm.at[pl.ds(off, CH)], rowv)
        pltpu.sync_copy(p_hbm.at[pl.ds(off, CH)], bufp)
        pltpu.sync_copy(bufp, acc.at[rowv], add=True)
        return 0

    lax.fori_loop(0, NCHUNK, step, 0)
    plsc.subcore_barrier()
    pltpu.sync_copy(acc.at[pl.ds(rbase, ROWS_PER_TILE)],
                    q_hbm.at[cid, pl.ds(rbase, ROWS_PER_TILE)])


def _full(shape):
    return pl.BlockSpec(shape, lambda i: (0,) * len(shape))


def _silu(v):
    return v * jax.nn.sigmoid(v)


def _build_body(x_ref, pos_ref, emb_ref, wr_ref, wc_ref,
                h_ref, a_ref, b_ref):
    ids = lax.broadcasted_iota(jnp.int32, (NB, 100), 1)
    oh = (x_ref[...] == ids).astype(jnp.float32)
    h = jnp.dot(oh, emb_ref[...], preferred_element_type=jnp.float32)
    posp = pos_ref[...]
    h_ref[...] = h
    a_ref[...] = jnp.concatenate(
        [jnp.dot(h, wr_ref[...], preferred_element_type=jnp.float32), posp], 1)
    b_ref[...] = jnp.concatenate(
        [jnp.dot(h, wc_ref[...], preferred_element_type=jnp.float32), posp], 1)


_build_tables = pl.pallas_call(
    _build_body,
    grid=(GRID_N,),
    in_specs=[
        pl.BlockSpec((NB, 1), lambda i: (i, 0)),
        pl.BlockSpec((NB, 16), lambda i: (i, 0)),
        _full((100, H)),
        _full((H, H)),
        _full((H, H)),
    ],
    out_specs=[
        pl.BlockSpec((NB, H), lambda i: (i, 0)),
        pl.BlockSpec((NB, W_TAB), lambda i: (i, 0)),
        pl.BlockSpec((NB, W_TAB), lambda i: (i, 0)),
    ],
    out_shape=[
        jax.ShapeDtypeStruct((N, H), jnp.float32),
        jax.ShapeDtypeStruct((N, W_TAB), jnp.float32),
        jax.ShapeDtypeStruct((N, W_TAB), jnp.float32),
    ],
)


def _edge_body(sa_ref, sb_ref, ea_ref, wa_ref, wd_ref, b1_ref,
               ew2_ref, b2_ref, cwr_ref, cb_ref, p_ref):
    sa = sa_ref[...]
    sb = sb_ref[...]
    hrow = sa[:, :H]
    hcol = sb[:, :H]
    posr = sa[:, H:]
    posc = sb[:, H:]
    rij = posr - posc
    dij = jnp.sum(rij * rij, axis=-1, keepdims=True)
    pre = (hrow + hcol
           + jnp.dot(ea_ref[...], wa_ref[...],
                     preferred_element_type=jnp.float32)
           + dij * wd_ref[...] + b1_ref[...])
    m1 = _silu(pre)
    m = _silu(jnp.dot(m1, ew2_ref[...],
                      preferred_element_type=jnp.float32) + b2_ref[...])
    w = _silu(jnp.sum(m * cwr_ref[...], axis=-1, keepdims=True) + cb_ref[...])
    rn = rij / (jnp.sqrt(dij) + 1e-8)
    p_ref[...] = jnp.concatenate([m, rn * w], 1)


_edge_mlp = pl.pallas_call(
    _edge_body,
    grid=(GRID_E,),
    in_specs=[
        pl.BlockSpec((EB, W_TAB), lambda i: (i, 0)),
        pl.BlockSpec((EB, W_TAB), lambda i: (i, 0)),
        pl.BlockSpec((EB, RBF), lambda i: (i, 0)),
        _full((RBF, H)),
        _full((1, H)),
        _full((1, H)),
        _full((H, H)),
        _full((1, H)),
        _full((1, H)),
        _full((1, 1)),
    ],
    out_specs=pl.BlockSpec((EB, W_TAB), lambda i: (i, 0)),
    out_shape=jax.ShapeDtypeStruct((E, W_TAB), jnp.float32),
)


def _node_body(h_ref, pos_ref, q_ref, w1a_ref, w1b_ref, b1_ref,
               w2_ref, b2_ref, wr_ref, wc_ref,
               h_out, pos_out, a_ref, b_ref):
    h = h_ref[...]
    q = q_ref[0] + q_ref[1]
    agg = q[:, :H]
    dpos = q[:, H:]
    nh = _silu(jnp.dot(h, w1a_ref[...], preferred_element_type=jnp.float32)
               + jnp.dot(agg, w1b_ref[...], preferred_element_type=jnp.float32)
               + b1_ref[...])
    hn = h + jnp.dot(nh, w2_ref[...],
                     preferred_element_type=jnp.float32) + b2_ref[...]
    posn = pos_ref[...] + dpos
    h_out[...] = hn
    pos_out[...] = posn
    a_ref[...] = jnp.concatenate(
        [jnp.dot(hn, wr_ref[...], preferred_element_type=jnp.float32), posn], 1)
    b_ref[...] = jnp.concatenate(
        [jnp.dot(hn, wc_ref[...], preferred_element_type=jnp.float32), posn], 1)


_node_update = pl.pallas_call(
    _node_body,
    grid=(GRID_N,),
    in_specs=[
        pl.BlockSpec((NB, H), lambda i: (i, 0)),
        pl.BlockSpec((NB, 16), lambda i: (i, 0)),
        pl.BlockSpec((NC, NB, W_TAB), lambda i: (0, i, 0)),
        _full((H, H)),
        _full((H, H)),
        _full((1, H)),
        _full((H, H)),
        _full((1, H)),
        _full((H, H)),
        _full((H, H)),
    ],
    out_specs=[
        pl.BlockSpec((NB, H), lambda i: (i, 0)),
        pl.BlockSpec((NB, 16), lambda i: (i, 0)),
        pl.BlockSpec((NB, W_TAB), lambda i: (i, 0)),
        pl.BlockSpec((NB, W_TAB), lambda i: (i, 0)),
    ],
    out_shape=[
        jax.ShapeDtypeStruct((N, H), jnp.float32),
        jax.ShapeDtypeStruct((N, 16), jnp.float32),
        jax.ShapeDtypeStruct((N, W_TAB), jnp.float32),
        jax.ShapeDtypeStruct((N, W_TAB), jnp.float32),
    ],
)


def _node_last_body(h_ref, q_ref, w1a_ref, w1b_ref, b1_ref,
                    w2_ref, b2_ref, h_out):
    h = h_ref[...]
    q = q_ref[0] + q_ref[1]
    agg = q[:, :H]
    nh = _silu(jnp.dot(h, w1a_ref[...], preferred_element_type=jnp.float32)
               + jnp.dot(agg, w1b_ref[...], preferred_element_type=jnp.float32)
               + b1_ref[...])
    h_out[...] = h + jnp.dot(nh, w2_ref[...],
                             preferred_element_type=jnp.float32) + b2_ref[...]


_node_update_last = pl.pallas_call(
    _node_last_body,
    grid=(GRID_N,),
    in_specs=[
        pl.BlockSpec((NB, H), lambda i: (i, 0)),
        pl.BlockSpec((NC, NB, W_TAB), lambda i: (0, i, 0)),
        _full((H, H)),
        _full((H, H)),
        _full((1, H)),
        _full((H, H)),
        _full((1, H)),
    ],
    out_specs=pl.BlockSpec((NB, H), lambda i: (i, 0)),
    out_shape=jax.ShapeDtypeStruct((N, H), jnp.float32),
)


def _final_body(h_ref, b_ref, lw_ref, lb_ref, out_ref, acc, cnt):
    i = pl.program_id(0)

    @pl.when(i == 0)
    def _():
        acc[...] = jnp.zeros_like(acc)
        cnt[...] = jnp.zeros_like(cnt)

    ids = lax.broadcasted_iota(jnp.int32, (NB, G), 1)
    oh = (b_ref[...] == ids).astype(jnp.float32)
    acc[...] += lax.dot_general(oh, h_ref[...], (((0,), (0,)), ((), ())),
                                preferred_element_type=jnp.float32)
    cnt[...] += jnp.sum(oh, axis=0, keepdims=True)

    @pl.when(i == GRID_N - 1)
    def _():
        mean = acc[...] / jnp.maximum(cnt[...].reshape(G, 1), 1.0)
        out_ref[...] = jnp.dot(jnp.maximum(mean, 0.0), lw_ref[...],
                               preferred_element_type=jnp.float32) + lb_ref[...]


_final_pool = pl.pallas_call(
    _final_body,
    grid=(GRID_N,),
    in_specs=[
        pl.BlockSpec((NB, H), lambda i: (i, 0)),
        pl.BlockSpec((NB, 1), lambda i: (i, 0)),
        _full((H, H)),
        _full((1, H)),
    ],
    out_specs=pl.BlockSpec((G, H), lambda i: (0, 0)),
    out_shape=jax.ShapeDtypeStruct((G, H), jnp.float32),
    scratch_shapes=[
        pltpu.VMEM((G, H), jnp.float32),
        pltpu.VMEM((1, G), jnp.float32),
    ],
    compiler_params=pltpu.CompilerParams(
        dimension_semantics=("arbitrary",)),
)


def kernel(x, pos, edge_index, edge_attr, batch, params):
    row = edge_index[0].astype(jnp.int32)
    col = edge_index[1].astype(jnp.int32)
    x2 = x.astype(jnp.int32).reshape(N, 1)
    batch2 = batch.astype(jnp.int32).reshape(N, 1)
    posp = jnp.pad(pos.astype(jnp.float32), ((0, 0), (0, 13)))
    zeros_nt = jnp.zeros((N, W_TAB), jnp.float32)

    layers = params["layers"]
    sliced = []
    for p in layers:
        sliced.append(dict(
            wr=p["e_w1"][:H],
            wc=p["e_w1"][H:2 * H],
            wd=p["e_w1"][2 * H:2 * H + 1],
            wa=p["e_w1"][2 * H + 1:],
            b1=p["e_b1"].reshape(1, H),
            ew2=p["e_w2"],
            b2=p["e_b2"].reshape(1, H),
            cwr=p["c_w"].reshape(1, H),
            cb=p["c_b"].reshape(1, 1),
            w1a=p["n_w1"][:H],
            w1b=p["n_w1"][H:],
            nb1=p["n_b1"].reshape(1, H),
            w2=p["n_w2"],
            nb2=p["n_b2"].reshape(1, H),
        ))

    h, A, B = _build_tables(x2, posp, params["emb"],
                            sliced[0]["wr"], sliced[0]["wc"])
    for li, s in enumerate(sliced):
        sa, sb = _sc_gather(A, B, row, col)
        pay = _edge_mlp(sa, sb, edge_attr, s["wa"], s["wd"], s["b1"],
                        s["ew2"], s["b2"], s["cwr"], s["cb"])
        q = _sc_scatter(pay, row, zeros_nt)
        if li + 1 < len(sliced):
            nxt = sliced[li + 1]
            h, posp, A, B = _node_update(h, posp, q,
                                         s["w1a"], s["w1b"], s["nb1"],
                                         s["w2"], s["nb2"],
                                         nxt["wr"], nxt["wc"])
        else:
            h = _node_update_last(h, q, s["w1a"], s["w1b"], s["nb1"],
                                  s["w2"], s["nb2"])

    return _final_pool(h, batch2, params["lin_w"],
                       params["lin_b"].reshape(1, H))


# speedup vs baseline: 2.6947x; 1.5661x over previous
import functools

import jax
import jax.numpy as jnp
from jax import lax
from jax.experimental import pallas as pl
from jax.experimental.pallas import tpu as pltpu
from jax.experimental.pallas import tpu_sc as plsc

N = 10000
E = 320000
H = 128
RBF = 16
G = 64

NC = 2
NS = 16
NW = NC * NS
EPW = E // NW
CH = 80
NCHUNK = EPW // CH
NPAD = 10240
CHP = 400
ZROWS = 80

NB = 1000
EB = 2560
GRID_N = N // NB
GRID_E = E // EB

_mesh = plsc.VectorSubcoreMesh(
    core_axis_name="c", subcore_axis_name="s", num_cores=NC, num_subcores=NS)

_f32 = jnp.float32


def _zero16():
    return jnp.zeros((16,), _f32)


def _iota16():
    return lax.iota(jnp.int32, 16)


@functools.partial(
    pl.kernel,
    mesh=_mesh,
    out_type=[jax.ShapeDtypeStruct((E, H), _f32),
              jax.ShapeDtypeStruct((E, H), _f32),
              jax.ShapeDtypeStruct((E * 4,), _f32)],
    scratch_types=[
        pltpu.VMEM((CH,), jnp.int32),
        pltpu.VMEM((CH,), jnp.int32),
        pltpu.VMEM((N * 4,), _f32),
        pltpu.VMEM((CH, H), _f32),
        pltpu.VMEM((CH, H), _f32),
        pltpu.VMEM((CH * 4,), _f32),
        pltpu.SemaphoreType.DMA,
        pltpu.SemaphoreType.DMA,
    ],
    compiler_params=pltpu.CompilerParams(needs_layout_passes=False),
)
def _sc_gather(ah_hbm, bh_hbm, posf_hbm, row_hbm, col_hbm,
               sa_hbm, sb_hbm, rijf_hbm,
               rowch, colch, posv, bufa, bufb, rbuf, sema, semb):
    wid = lax.axis_index("s") * NC + lax.axis_index("c")
    base = wid * EPW
    pltpu.sync_copy(posf_hbm, posv)
    for z in range(CH * 4 // 16):
        rbuf[pl.ds(z * 16, 16)] = _zero16()

    def step(k, _):
        off = base + k * CH
        pltpu.sync_copy(row_hbm.at[pl.ds(off, CH)], rowch)
        pltpu.sync_copy(col_hbm.at[pl.ds(off, CH)], colch)
        ca = pltpu.async_copy(ah_hbm.at[rowch], bufa, sema)
        cb = pltpu.async_copy(bh_hbm.at[colch], bufb, semb)
        for g in range(CH // 16):
            rv = rowch[pl.ds(g * 16, 16)]
            cv = colch[pl.ds(g * 16, 16)]
            rv4 = rv * 4
            cv4 = cv * 4
            for c3 in range(3):
                xr = plsc.load_gather(posv, [rv4 + c3])
                xc = plsc.load_gather(posv, [cv4 + c3])
                plsc.store_scatter(rbuf, [_iota16() * 4 + (g * 64 + c3)],
                                   xr - xc)
        ca.wait()
        cb.wait()
        pltpu.sync_copy(bufa, sa_hbm.at[pl.ds(off, CH)])
        pltpu.sync_copy(bufb, sb_hbm.at[pl.ds(off, CH)])
        pltpu.sync_copy(rbuf, rijf_hbm.at[pl.ds(off * 4, CH * 4)])
        return 0

    lax.fori_loop(0, NCHUNK, step, 0)


@functools.partial(
    pl.kernel,
    mesh=_mesh,
    out_type=jax.ShapeDtypeStruct((NC, NPAD, H), _f32),
    scratch_types=[
        pltpu.VMEM((CH,), jnp.int32),
        pltpu.VMEM((CH, H), _f32),
        pltpu.VMEM((ZROWS, H), _f32),
        pltpu.VMEM_SHARED((NPAD, H), _f32),
        pltpu.SemaphoreType.DMA,
    ],
)
def _sc_scatter(p_hbm, row_hbm, q_hbm, rowch, bufp, zbuf, acc, psem):
    cid = lax.axis_index("c")
    sid = lax.axis_index("s")
    wid = sid * NC + cid
    base = wid * EPW
    rbase = sid * (NPAD // NS)

    def zrow(r, _):
        for l8 in range(H // 16):
            zbuf[r, pl.ds(l8 * 16, 16)] = _zero16()
        return 0

    lax.fori_loop(0, ZROWS, zrow, 0)
    for t in range(NPAD // NS // ZROWS):
        pltpu.sync_copy(zbuf, acc.at[pl.ds(rbase + t * ZROWS, ZROWS)])
    plsc.subcore_barrier()

    def step(k, _):
        off = base + k * CH
        pltpu.sync_copy(row_hbm.at[pl.ds(off, CH)], rowch)
        pltpu.async_copy(p_hbm.at[pl.ds(off, CH)], bufp, psem).wait()
        pltpu.sync_copy(bufp, acc.at[rowch], add=True)
        return 0

    lax.fori_loop(0, NCHUNK, step, 0)
    plsc.subcore_barrier()
    for t in range(NPAD // NS // ZROWS):
        pltpu.sync_copy(acc.at[pl.ds(rbase + t * ZROWS, ZROWS)],
                        q_hbm.at[cid, pl.ds(rbase + t * ZROWS, ZROWS)])


@functools.partial(
    pl.kernel,
    mesh=_mesh,
    out_type=jax.ShapeDtypeStruct((NW * NPAD * 4,), _f32),
    scratch_types=[
        pltpu.VMEM((CHP,), jnp.int32),
        pltpu.VMEM((CHP * 4,), _f32),
        pltpu.VMEM((NPAD * 4,), _f32),
    ],
    compiler_params=pltpu.CompilerParams(use_tc_tiling_on_sc=False,
                                         needs_layout_passes=False),
)
def _sc_scatter_pos(rnwf_hbm, row_hbm, o_hbm, rowch, rnwch, acc2):
    wid = lax.axis_index("s") * NC + lax.axis_index("c")
    base = wid * EPW

    def zstep(j, _):
        acc2[pl.ds(j * 16, 16)] = _zero16()
        return 0

    lax.fori_loop(0, NPAD * 4 // 16, zstep, 0)

    def step(k, _):
        off = base + k * CHP
        pltpu.sync_copy(row_hbm.at[pl.ds(off, CHP)], rowch)
        pltpu.sync_copy(rnwf_hbm.at[pl.ds(off * 4, CHP * 4)], rnwch)
        for g in range(CHP // 16):
            rv4 = rowch[pl.ds(g * 16, 16)] * 4
            for c3 in range(3):
                vals = plsc.load_gather(rnwch,
                                        [_iota16() * 4 + (g * 64 + c3)])
                plsc.addupdate_scatter(acc2, [rv4 + c3], vals)
        return 0

    lax.fori_loop(0, EPW // CHP, step, 0)
    pltpu.sync_copy(acc2, o_hbm.at[pl.ds(wid * (NPAD * 4), NPAD * 4)])


def _full(shape):
    return pl.BlockSpec(shape, lambda i: (0,) * len(shape))


def _silu(v):
    return v * jax.nn.sigmoid(v)


def _build_body(x_ref, emb_ref, wr_ref, wc_ref, h_ref, a_ref, b_ref):
    ids = lax.broadcasted_iota(jnp.int32, (NB, 100), 1)
    oh = (x_ref[...] == ids).astype(_f32)
    h = jnp.dot(oh, emb_ref[...], preferred_element_type=_f32)
    h_ref[...] = h
    a_ref[...] = jnp.dot(h, wr_ref[...], preferred_element_type=_f32)
    b_ref[...] = jnp.dot(h, wc_ref[...], preferred_element_type=_f32)


_build_tables = pl.pallas_call(
    _build_body,
    grid=(GRID_N,),
    in_specs=[
        pl.BlockSpec((NB, 1), lambda i: (i, 0)),
        _full((100, H)),
        _full((H, H)),
        _full((H, H)),
    ],
    out_specs=[
        pl.BlockSpec((NB, H), lambda i: (i, 0)),
        pl.BlockSpec((NB, H), lambda i: (i, 0)),
        pl.BlockSpec((NB, H), lambda i: (i, 0)),
    ],
    out_shape=[
        jax.ShapeDtypeStruct((N, H), _f32),
        jax.ShapeDtypeStruct((N, H), _f32),
        jax.ShapeDtypeStruct((N, H), _f32),
    ],
)


def _edge_body(sa_ref, sb_ref, rij_ref, ea_ref, wa_ref, wd_ref, b1_ref,
               ew2_ref, b2_ref, cwr_ref, cb_ref, m_ref, rnw_ref):
    rij = rij_ref[...]
    dij = jnp.sum(rij * rij, axis=-1, keepdims=True)
    pre = (sa_ref[...] + sb_ref[...]
           + jnp.dot(ea_ref[...], wa_ref[...], preferred_element_type=_f32)
           + dij * wd_ref[...] + b1_ref[...])
    m1 = _silu(pre)
    m = _silu(jnp.dot(m1, ew2_ref[...],
                      preferred_element_type=_f32) + b2_ref[...])
    w = _silu(jnp.sum(m * cwr_ref[...], axis=-1, keepdims=True) + cb_ref[...])
    rn = rij / (jnp.sqrt(dij) + 1e-8)
    m_ref[...] = m
    rnw_ref[...] = rn * w


_edge_mlp = pl.pallas_call(
    _edge_body,
    grid=(GRID_E,),
    in_specs=[
        pl.BlockSpec((EB, H), lambda i: (i, 0)),
        pl.BlockSpec((EB, H), lambda i: (i, 0)),
        pl.BlockSpec((EB, 4), lambda i: (i, 0)),
        pl.BlockSpec((EB, RBF), lambda i: (i, 0)),
        _full((RBF, H)),
        _full((1, H)),
        _full((1, H)),
        _full((H, H)),
        _full((1, H)),
        _full((1, H)),
        _full((1, 1)),
    ],
    out_specs=[
        pl.BlockSpec((EB, H), lambda i: (i, 0)),
        pl.BlockSpec((EB, 4), lambda i: (i, 0)),
    ],
    out_shape=[
        jax.ShapeDtypeStruct((E, H), _f32),
        jax.ShapeDtypeStruct((E, 4), _f32),
    ],
)


def _node_body(h_ref, pos_ref, q_ref, q2_ref, w1a_ref, w1b_ref, b1_ref,
               w2_ref, b2_ref, wr_ref, wc_ref,
               h_out, pos_out, a_ref, b_ref):
    h = h_ref[...]
    agg = q_ref[0] + q_ref[1]
    dpos = jnp.sum(q2_ref[...], axis=0)
    nh = _silu(jnp.dot(h, w1a_ref[...], preferred_element_type=_f32)
               + jnp.dot(agg, w1b_ref[...], preferred_element_type=_f32)
               + b1_ref[...])
    hn = h + jnp.dot(nh, w2_ref[...], preferred_element_type=_f32) + b2_ref[...]
    posn = pos_ref[...] + dpos
    h_out[...] = hn
    pos_out[...] = posn
    a_ref[...] = jnp.dot(hn, wr_ref[...], preferred_element_type=_f32)
    b_ref[...] = jnp.dot(hn, wc_ref[...], preferred_element_type=_f32)


_node_update = pl.pallas_call(
    _node_body,
    grid=(GRID_N,),
    in_specs=[
        pl.BlockSpec((NB, H), lambda i: (i, 0)),
        pl.BlockSpec((NB, 4), lambda i: (i, 0)),
        pl.BlockSpec((NC, NB, H), lambda i: (0, i, 0)),
        pl.BlockSpec((NW, NB, 4), lambda i: (0, i, 0)),
        _full((H, H)),
        _full((H, H)),
        _full((1, H)),
        _full((H, H)),
        _full((1, H)),
        _full((H, H)),
        _full((H, H)),
    ],
    out_specs=[
        pl.BlockSpec((NB, H), lambda i: (i, 0)),
        pl.BlockSpec((NB, 4), lambda i: (i, 0)),
        pl.BlockSpec((NB, H), lambda i: (i, 0)),
        pl.BlockSpec((NB, H), lambda i: (i, 0)),
    ],
    out_shape=[
        jax.ShapeDtypeStruct((N, H), _f32),
        jax.ShapeDtypeStruct((N, 4), _f32),
        jax.ShapeDtypeStruct((N, H), _f32),
        jax.ShapeDtypeStruct((N, H), _f32),
    ],
)


def _node_last_body(h_ref, q_ref, w1a_ref, w1b_ref, b1_ref,
                    w2_ref, b2_ref, h_out):
    h = h_ref[...]
    agg = q_ref[0] + q_ref[1]
    nh = _silu(jnp.dot(h, w1a_ref[...], preferred_element_type=_f32)
               + jnp.dot(agg, w1b_ref[...], preferred_element_type=_f32)
               + b1_ref[...])
    h_out[...] = h + jnp.dot(nh, w2_ref[...],
                             preferred_element_type=_f32) + b2_ref[...]


_node_update_last = pl.pallas_call(
    _node_last_body,
    grid=(GRID_N,),
    in_specs=[
        pl.BlockSpec((NB, H), lambda i: (i, 0)),
        pl.BlockSpec((NC, NB, H), lambda i: (0, i, 0)),
        _full((H, H)),
        _full((H, H)),
        _full((1, H)),
        _full((H, H)),
        _full((1, H)),
    ],
    out_specs=pl.BlockSpec((NB, H), lambda i: (i, 0)),
    out_shape=jax.ShapeDtypeStruct((N, H), _f32),
)


def _final_body(h_ref, b_ref, lw_ref, lb_ref, out_ref, acc, cnt):
    i = pl.program_id(0)

    @pl.when(i == 0)
    def _():
        acc[...] = jnp.zeros_like(acc)
        cnt[...] = jnp.zeros_like(cnt)

    ids = lax.broadcasted_iota(jnp.int32, (NB, G), 1)
    oh = (b_ref[...] == ids).astype(_f32)
    acc[...] += lax.dot_general(oh, h_ref[...], (((0,), (0,)), ((), ())),
                                preferred_element_type=_f32)
    cnt[...] += jnp.sum(oh, axis=0, keepdims=True)

    @pl.when(i == GRID_N - 1)
    def _():
        mean = acc[...] / jnp.maximum(cnt[...].reshape(G, 1), 1.0)
        out_ref[...] = jnp.dot(jnp.maximum(mean, 0.0), lw_ref[...],
                               preferred_element_type=_f32) + lb_ref[...]


_final_pool = pl.pallas_call(
    _final_body,
    grid=(GRID_N,),
    in_specs=[
        pl.BlockSpec((NB, H), lambda i: (i, 0)),
        pl.BlockSpec((NB, 1), lambda i: (i, 0)),
        _full((H, H)),
        _full((1, H)),
    ],
    out_specs=pl.BlockSpec((G, H), lambda i: (0, 0)),
    out_shape=jax.ShapeDtypeStruct((G, H), _f32),
    scratch_shapes=[
        pltpu.VMEM((G, H), _f32),
        pltpu.VMEM((1, G), _f32),
    ],
    compiler_params=pltpu.CompilerParams(
        dimension_semantics=("arbitrary",)),
)


def kernel(x, pos, edge_index, edge_attr, batch, params):
    row = edge_index[0].astype(jnp.int32)
    col = edge_index[1].astype(jnp.int32)
    x2 = x.astype(jnp.int32).reshape(N, 1)
    batch2 = batch.astype(jnp.int32).reshape(N, 1)
    pos4 = jnp.pad(pos.astype(_f32), ((0, 0), (0, 1)))

    layers = params["layers"]
    sliced = []
    for p in layers:
        sliced.append(dict(
            wr=p["e_w1"][:H],
            wc=p["e_w1"][H:2 * H],
            wd=p["e_w1"][2 * H:2 * H + 1],
            wa=p["e_w1"][2 * H + 1:],
            b1=p["e_b1"].reshape(1, H),
            ew2=p["e_w2"],
            b2=p["e_b2"].reshape(1, H),
            cwr=p["c_w"].reshape(1, H),
            cb=p["c_b"].reshape(1, 1),
            w1a=p["n_w1"][:H],
            w1b=p["n_w1"][H:],
            nb1=p["n_b1"].reshape(1, H),
            w2=p["n_w2"],
            nb2=p["n_b2"].reshape(1, H),
        ))

    h, ah, bh = _build_tables(x2, params["emb"],
                              sliced[0]["wr"], sliced[0]["wc"])
    for li, s in enumerate(sliced):
        posf = pos4.reshape(N * 4)
        sa, sb, rijf = _sc_gather(ah, bh, posf, row, col)
        rij4 = rijf.reshape(E, 4)
        m, rnw = _edge_mlp(sa, sb, rij4, edge_attr, s["wa"], s["wd"],
                           s["b1"], s["ew2"], s["b2"], s["cwr"], s["cb"])
        q = _sc_scatter(m, row)
        if li + 1 < len(sliced):
            rnwf = rnw.reshape(E * 4)
            q2f = _sc_scatter_pos(rnwf, row)
            q2 = q2f.reshape(NW, NPAD, 4)[:, :N]
            nxt = sliced[li + 1]
            h, pos4, ah, bh = _node_update(h, pos4, q, q2,
                                           s["w1a"], s["w1b"], s["nb1"],
                                           s["w2"], s["nb2"],
                                           nxt["wr"], nxt["wc"])
        else:
            h = _node_update_last(h, q, s["w1a"], s["w1b"], s["nb1"],
                                  s["w2"], s["nb2"])

    return _final_pool(h, batch2, params["lin_w"],
                       params["lin_b"].reshape(1, H))


# speedup vs baseline: 3.4459x; 1.2787x over previous
import functools

import jax
import jax.numpy as jnp
from jax import lax
from jax.experimental import pallas as pl
from jax.experimental.pallas import tpu as pltpu
from jax.experimental.pallas import tpu_sc as plsc

N = 10000
E = 320000
H = 128
RBF = 16
G = 64

NC = 2
NS = 16
NW = NC * NS
EPW = E // NW
CH = 80
NCHUNK = EPW // CH
NPAD = 10240
CHP = 400
ZROWS = 80

NB = 1000
EB = 2560
GRID_N = N // NB
GRID_E = E // EB

_mesh = plsc.VectorSubcoreMesh(
    core_axis_name="c", subcore_axis_name="s", num_cores=NC, num_subcores=NS)

_f32 = jnp.float32


def _zero16():
    return jnp.zeros((16,), _f32)


def _iota16():
    return lax.iota(jnp.int32, 16)


@functools.partial(
    pl.kernel,
    mesh=_mesh,
    out_type=[jax.ShapeDtypeStruct((E, H), _f32),
              jax.ShapeDtypeStruct((E * 4,), _f32)],
    scratch_types=(
        [pltpu.VMEM((N * 4,), _f32)]
        + 2 * [pltpu.VMEM((CH,), jnp.int32),
               pltpu.VMEM((CH,), jnp.int32),
               pltpu.VMEM((CH, H), _f32),
               pltpu.VMEM((CH, H), _f32),
               pltpu.VMEM((CH * 4,), _f32),
               pltpu.SemaphoreType.DMA,
               pltpu.SemaphoreType.DMA,
               pltpu.SemaphoreType.DMA]
    ),
    compiler_params=pltpu.CompilerParams(needs_layout_passes=False),
)
def _sc_gather(ah_hbm, bh_hbm, posf_hbm, row_hbm, col_hbm,
               s_hbm, rijf_hbm, posv, *bufs):
    wid = lax.axis_index("s") * NC + lax.axis_index("c")
    base = wid * EPW
    pltpu.sync_copy(posf_hbm, posv)
    sets = (bufs[0:8], bufs[8:16])
    for st in sets:
        rbuf = st[4]
        for z in range(CH * 4 // 16):
            rbuf[pl.ds(z * 16, 16)] = _zero16()

    def fire_idx(c, st):
        rowch, colch, isem = st[0], st[1], st[5]
        off = base + c * CH
        pltpu.async_copy(row_hbm.at[pl.ds(off, CH)], rowch, isem)
        pltpu.async_copy(col_hbm.at[pl.ds(off, CH)], colch, isem)

    def wait_idx(st):
        rowch, colch, isem = st[0], st[1], st[5]
        pltpu.make_async_copy(row_hbm.at[pl.ds(0, CH)], rowch, isem).wait()
        pltpu.make_async_copy(col_hbm.at[pl.ds(0, CH)], colch, isem).wait()

    def fire_gath(st):
        rowch, colch, bufa, bufb, gsem = st[0], st[1], st[2], st[3], st[6]
        pltpu.async_copy(ah_hbm.at[rowch], bufa, gsem)
        pltpu.async_copy(bh_hbm.at[colch], bufb, gsem)

    def wait_gath(st):
        bufa, bufb, gsem = st[2], st[3], st[6]
        pltpu.make_async_copy(ah_hbm.at[pl.ds(0, CH)], bufa, gsem).wait()
        pltpu.make_async_copy(bh_hbm.at[pl.ds(0, CH)], bufb, gsem).wait()

    def fire_wb(c, st):
        bufa, rbuf, wsem = st[2], st[4], st[7]
        off = base + c * CH
        pltpu.async_copy(bufa, s_hbm.at[pl.ds(off, CH)], wsem)
        pltpu.async_copy(rbuf, rijf_hbm.at[pl.ds(off * 4, CH * 4)], wsem)

    def wait_wb(st):
        bufa, rbuf, wsem = st[2], st[4], st[7]
        pltpu.make_async_copy(bufa, s_hbm.at[pl.ds(0, CH)], wsem).wait()
        pltpu.make_async_copy(rbuf, rijf_hbm.at[pl.ds(0, CH * 4)],
                              wsem).wait()

    def compute(st):
        rowch, colch, bufa, bufb, rbuf = st[0], st[1], st[2], st[3], st[4]

        def addrow(r, _):
            for l8 in range(H // 16):
                sl = pl.ds(l8 * 16, 16)
                bufa[r, sl] = bufa[r, sl] + bufb[r, sl]
            return 0

        lax.fori_loop(0, CH, addrow, 0)
        for g in range(CH // 16):
            rv4 = rowch[pl.ds(g * 16, 16)] * 4
            cv4 = colch[pl.ds(g * 16, 16)] * 4
            for c3 in range(3):
                xr = plsc.load_gather(posv, [rv4 + c3])
                xc = plsc.load_gather(posv, [cv4 + c3])
                plsc.store_scatter(rbuf, [_iota16() * 4 + (g * 64 + c3)],
                                   xr - xc)

    def step(c, p):
        sp, sq = sets[p], sets[1 - p]
        wait_gath(sp)
        wait_idx(sq)

        @pl.when(c >= 1)
        def _():
            wait_wb(sq)

        fire_gath(sq)
        compute(sp)

        @pl.when(c + 2 <= NCHUNK - 1)
        def _():
            fire_idx(c + 2, sp)

        fire_wb(c, sp)

    fire_idx(0, sets[0])
    fire_idx(1, sets[1])
    wait_idx(sets[0])
    fire_gath(sets[0])

    def pair(j, _):
        step(2 * j, 0)
        step(2 * j + 1, 1)
        return 0

    lax.fori_loop(0, (NCHUNK - 1) // 2, pair, 0)
    c_last = NCHUNK - 1
    sp, sq = sets[0], sets[1]
    wait_gath(sp)
    wait_wb(sq)
    compute(sp)
    fire_wb(c_last, sp)
    wait_wb(sp)


@functools.partial(
    pl.kernel,
    mesh=_mesh,
    out_type=jax.ShapeDtypeStruct((NC, NPAD, H), _f32),
    scratch_types=(
        [pltpu.VMEM((ZROWS, H), _f32),
         pltpu.VMEM_SHARED((NPAD, H), _f32)]
        + [pltpu.VMEM((CH,), jnp.int32),
           pltpu.VMEM((CH, H), _f32),
           pltpu.SemaphoreType.DMA]
        + [pltpu.VMEM((CH,), jnp.int32),
           pltpu.VMEM((CH, H), _f32),
           pltpu.SemaphoreType.DMA]
    ),
    compiler_params=pltpu.CompilerParams(needs_layout_passes=False),
)
def _sc_scatter(p_hbm, row_hbm, q_hbm, zbuf, acc, *bufs):
    cid = lax.axis_index("c")
    sid = lax.axis_index("s")
    wid = sid * NC + cid
    base = wid * EPW
    rbase = sid * (NPAD // NS)
    sets = (bufs[0:3], bufs[3:6])

    def fire(c, st):
        rowch, bufp, sem = st
        off = base + c * CH
        pltpu.async_copy(row_hbm.at[pl.ds(off, CH)], rowch, sem)
        pltpu.async_copy(p_hbm.at[pl.ds(off, CH)], bufp, sem)

    def wait(st):
        rowch, bufp, sem = st
        pltpu.make_async_copy(row_hbm.at[pl.ds(0, CH)], rowch, sem).wait()
        pltpu.make_async_copy(p_hbm.at[pl.ds(0, CH)], bufp, sem).wait()

    fire(0, sets[0])
    fire(1, sets[1])

    def zrow(r, _):
        for l8 in range(H // 16):
            zbuf[r, pl.ds(l8 * 16, 16)] = _zero16()
        return 0

    lax.fori_loop(0, ZROWS, zrow, 0)
    for t in range(NPAD // NS // ZROWS):
        pltpu.sync_copy(zbuf, acc.at[pl.ds(rbase + t * ZROWS, ZROWS)])
    plsc.subcore_barrier()

    def step(c, p):
        st = sets[p]
        wait(st)
        pltpu.sync_copy(st[1], acc.at[st[0]], add=True)

        @pl.when(c + 2 <= NCHUNK - 1)
        def _():
            fire(c + 2, st)

    def pair(j, _):
        step(2 * j, 0)
        step(2 * j + 1, 1)
        return 0

    lax.fori_loop(0, (NCHUNK - 1) // 2, pair, 0)
    step(NCHUNK - 1, 0)
    plsc.subcore_barrier()
    for t in range(NPAD // NS // ZROWS):
        pltpu.sync_copy(acc.at[pl.ds(rbase + t * ZROWS, ZROWS)],
                        q_hbm.at[cid, pl.ds(rbase + t * ZROWS, ZROWS)])


@functools.partial(
    pl.kernel,
    mesh=_mesh,
    out_type=jax.ShapeDtypeStruct((NW * NPAD * 4,), _f32),
    scratch_types=[
        pltpu.VMEM((CHP,), jnp.int32),
        pltpu.VMEM((CHP * 4,), _f32),
        pltpu.VMEM((NPAD * 4,), _f32),
    ],
    compiler_params=pltpu.CompilerParams(use_tc_tiling_on_sc=False,
                                         needs_layout_passes=False),
)
def _sc_scatter_pos(rnwf_hbm, row_hbm, o_hbm, rowch, rnwch, acc2):
    wid = lax.axis_index("s") * NC + lax.axis_index("c")
    base = wid * EPW

    def zstep(j, _):
        acc2[pl.ds(j * 16, 16)] = _zero16()
        return 0

    lax.fori_loop(0, NPAD * 4 // 16, zstep, 0)

    def step(k, _):
        off = base + k * CHP
        pltpu.sync_copy(row_hbm.at[pl.ds(off, CHP)], rowch)
        pltpu.sync_copy(rnwf_hbm.at[pl.ds(off * 4, CHP * 4)], rnwch)
        for g in range(CHP // 16):
            rv4 = rowch[pl.ds(g * 16, 16)] * 4
            for c3 in range(3):
                vals = plsc.load_gather(rnwch,
                                        [_iota16() * 4 + (g * 64 + c3)])
                plsc.addupdate_scatter(acc2, [rv4 + c3], vals)
        return 0

    lax.fori_loop(0, EPW // CHP, step, 0)
    pltpu.sync_copy(acc2, o_hbm.at[pl.ds(wid * (NPAD * 4), NPAD * 4)])


def _full(shape):
    return pl.BlockSpec(shape, lambda i: (0,) * len(shape))


def _silu(v):
    return v * jax.nn.sigmoid(v)


def _build_body(x_ref, emb_ref, wr_ref, wc_ref, h_ref, a_ref, b_ref):
    ids = lax.broadcasted_iota(jnp.int32, (NB, 100), 1)
    oh = (x_ref[...] == ids).astype(_f32)
    h = jnp.dot(oh, emb_ref[...], preferred_element_type=_f32)
    h_ref[...] = h
    a_ref[...] = jnp.dot(h, wr_ref[...], preferred_element_type=_f32)
    b_ref[...] = jnp.dot(h, wc_ref[...], preferred_element_type=_f32)


_build_tables = pl.pallas_call(
    _build_body,
    grid=(GRID_N,),
    in_specs=[
        pl.BlockSpec((NB, 1), lambda i: (i, 0)),
        _full((100, H)),
        _full((H, H)),
        _full((H, H)),
    ],
    out_specs=[
        pl.BlockSpec((NB, H), lambda i: (i, 0)),
        pl.BlockSpec((NB, H), lambda i: (i, 0)),
        pl.BlockSpec((NB, H), lambda i: (i, 0)),
    ],
    out_shape=[
        jax.ShapeDtypeStruct((N, H), _f32),
        jax.ShapeDtypeStruct((N, H), _f32),
        jax.ShapeDtypeStruct((N, H), _f32),
    ],
)


def _edge_body(s_ref, rij_ref, ea_ref, wa_ref, wd_ref, b1_ref,
               ew2_ref, b2_ref, cwr_ref, cb_ref, m_ref, rnw_ref):
    rij = rij_ref[...]
    dij = jnp.sum(rij * rij, axis=-1, keepdims=True)
    pre = (s_ref[...]
           + jnp.dot(ea_ref[...], wa_ref[...], preferred_element_type=_f32)
           + dij * wd_ref[...] + b1_ref[...])
    m1 = _silu(pre)
    m = _silu(jnp.dot(m1, ew2_ref[...],
                      preferred_element_type=_f32) + b2_ref[...])
    w = _silu(jnp.sum(m * cwr_ref[...], axis=-1, keepdims=True) + cb_ref[...])
    rn = rij / (jnp.sqrt(dij) + 1e-8)
    m_ref[...] = m
    rnw_ref[...] = rn * w


_edge_mlp = pl.pallas_call(
    _edge_body,
    grid=(GRID_E,),
    in_specs=[
        pl.BlockSpec((EB, H), lambda i: (i, 0)),
        pl.BlockSpec((EB, 4), lambda i: (i, 0)),
        pl.BlockSpec((EB, RBF), lambda i: (i, 0)),
        _full((RBF, H)),
        _full((1, H)),
        _full((1, H)),
        _full((H, H)),
        _full((1, H)),
        _full((1, H)),
        _full((1, 1)),
    ],
    out_specs=[
        pl.BlockSpec((EB, H), lambda i: (i, 0)),
        pl.BlockSpec((EB, 4), lambda i: (i, 0)),
    ],
    out_shape=[
        jax.ShapeDtypeStruct((E, H), _f32),
        jax.ShapeDtypeStruct((E, 4), _f32),
    ],
)


def _node_body(h_ref, pos_ref, q_ref, q2_ref, w1a_ref, w1b_ref, b1_ref,
               w2_ref, b2_ref, wr_ref, wc_ref,
               h_out, pos_out, a_ref, b_ref):
    h = h_ref[...]
    agg = q_ref[0] + q_ref[1]
    dpos = jnp.sum(q2_ref[...], axis=0)
    nh = _silu(jnp.dot(h, w1a_ref[...], preferred_element_type=_f32)
               + jnp.dot(agg, w1b_ref[...], preferred_element_type=_f32)
               + b1_ref[...])
    hn = h + jnp.dot(nh, w2_ref[...], preferred_element_type=_f32) + b2_ref[...]
    posn = pos_ref[...] + dpos
    h_out[...] = hn
    pos_out[...] = posn
    a_ref[...] = jnp.dot(hn, wr_ref[...], preferred_element_type=_f32)
    b_ref[...] = jnp.dot(hn, wc_ref[...], preferred_element_type=_f32)


_node_update = pl.pallas_call(
    _node_body,
    grid=(GRID_N,),
    in_specs=[
        pl.BlockSpec((NB, H), lambda i: (i, 0)),
        pl.BlockSpec((NB, 4), lambda i: (i, 0)),
        pl.BlockSpec((NC, NB, H), lambda i: (0, i, 0)),
        pl.BlockSpec((NW, NB, 4), lambda i: (0, i, 0)),
        _full((H, H)),
        _full((H, H)),
        _full((1, H)),
        _full((H, H)),
        _full((1, H)),
        _full((H, H)),
        _full((H, H)),
    ],
    out_specs=[
        pl.BlockSpec((NB, H), lambda i: (i, 0)),
        pl.BlockSpec((NB, 4), lambda i: (i, 0)),
        pl.BlockSpec((NB, H), lambda i: (i, 0)),
        pl.BlockSpec((NB, H), lambda i: (i, 0)),
    ],
    out_shape=[
        jax.ShapeDtypeStruct((N, H), _f32),
        jax.ShapeDtypeStruct((N, 4), _f32),
        jax.ShapeDtypeStruct((N, H), _f32),
        jax.ShapeDtypeStruct((N, H), _f32),
    ],
)


def _node_last_body(h_ref, q_ref, w1a_ref, w1b_ref, b1_ref,
                    w2_ref, b2_ref, h_out):
    h = h_ref[...]
    agg = q_ref[0] + q_ref[1]
    nh = _silu(jnp.dot(h, w1a_ref[...], preferred_element_type=_f32)
               + jnp.dot(agg, w1b_ref[...], preferred_element_type=_f32)
               + b1_ref[...])
    h_out[...] = h + jnp.dot(nh, w2_ref[...],
                             preferred_element_type=_f32) + b2_ref[...]


_node_update_last = pl.pallas_call(
    _node_last_body,
    grid=(GRID_N,),
    in_specs=[
        pl.BlockSpec((NB, H), lambda i: (i, 0)),
        pl.BlockSpec((NC, NB, H), lambda i: (0, i, 0)),
        _full((H, H)),
        _full((H, H)),
        _full((1, H)),
        _full((H, H)),
        _full((1, H)),
    ],
    out_specs=pl.BlockSpec((NB, H), lambda i: (i, 0)),
    out_shape=jax.ShapeDtypeStruct((N, H), _f32),
)


def _final_body(h_ref, b_ref, lw_ref, lb_ref, out_ref, acc, cnt):
    i = pl.program_id(0)

    @pl.when(i == 0)
    def _():
        acc[...] = jnp.zeros_like(acc)
        cnt[...] = jnp.zeros_like(cnt)

    ids = lax.broadcasted_iota(jnp.int32, (NB, G), 1)
    oh = (b_ref[...] == ids).astype(_f32)
    acc[...] += lax.dot_general(oh, h_ref[...], (((0,), (0,)), ((), ())),
                                preferred_element_type=_f32)
    cnt[...] += jnp.sum(oh, axis=0, keepdims=True)

    @pl.when(i == GRID_N - 1)
    def _():
        mean = acc[...] / jnp.maximum(cnt[...].reshape(G, 1), 1.0)
        out_ref[...] = jnp.dot(jnp.maximum(mean, 0.0), lw_ref[...],
                               preferred_element_type=_f32) + lb_ref[...]


_final_pool = pl.pallas_call(
    _final_body,
    grid=(GRID_N,),
    in_specs=[
        pl.BlockSpec((NB, H), lambda i: (i, 0)),
        pl.BlockSpec((NB, 1), lambda i: (i, 0)),
        _full((H, H)),
        _full((1, H)),
    ],
    out_specs=pl.BlockSpec((G, H), lambda i: (0, 0)),
    out_shape=jax.ShapeDtypeStruct((G, H), _f32),
    scratch_shapes=[
        pltpu.VMEM((G, H), _f32),
        pltpu.VMEM((1, G), _f32),
    ],
    compiler_params=pltpu.CompilerParams(
        dimension_semantics=("arbitrary",)),
)


def kernel(x, pos, edge_index, edge_attr, batch, params):
    row = edge_index[0].astype(jnp.int32)
    col = edge_index[1].astype(jnp.int32)
    x2 = x.astype(jnp.int32).reshape(N, 1)
    batch2 = batch.astype(jnp.int32).reshape(N, 1)
    pos4 = jnp.pad(pos.astype(_f32), ((0, 0), (0, 1)))

    layers = params["layers"]
    sliced = []
    for p in layers:
        sliced.append(dict(
            wr=p["e_w1"][:H],
            wc=p["e_w1"][H:2 * H],
            wd=p["e_w1"][2 * H:2 * H + 1],
            wa=p["e_w1"][2 * H + 1:],
            b1=p["e_b1"].reshape(1, H),
            ew2=p["e_w2"],
            b2=p["e_b2"].reshape(1, H),
            cwr=p["c_w"].reshape(1, H),
            cb=p["c_b"].reshape(1, 1),
            w1a=p["n_w1"][:H],
            w1b=p["n_w1"][H:],
            nb1=p["n_b1"].reshape(1, H),
            w2=p["n_w2"],
            nb2=p["n_b2"].reshape(1, H),
        ))

    h, ah, bh = _build_tables(x2, params["emb"],
                              sliced[0]["wr"], sliced[0]["wc"])
    for li, s in enumerate(sliced):
        posf = pos4.reshape(N * 4)
        ssum, rijf = _sc_gather(ah, bh, posf, row, col)
        rij4 = rijf.reshape(E, 4)
        m, rnw = _edge_mlp(ssum, rij4, edge_attr, s["wa"], s["wd"],
                           s["b1"], s["ew2"], s["b2"], s["cwr"], s["cb"])
        q = _sc_scatter(m, row)
        if li + 1 < len(sliced):
            rnwf = rnw.reshape(E * 4)
            q2f = _sc_scatter_pos(rnwf, row)
            q2 = q2f.reshape(NW, NPAD, 4)[:, :N]
            nxt = sliced[li + 1]
            h, pos4, ah, bh = _node_update(h, pos4, q, q2,
                                           s["w1a"], s["w1b"], s["nb1"],
                                           s["w2"], s["nb2"],
                                           nxt["wr"], nxt["wc"])
        else:
            h = _node_update_last(h, q, s["w1a"], s["w1b"], s["nb1"],
                                  s["w2"], s["nb2"])

    return _final_pool(h, batch2, params["lin_w"],
                       params["lin_b"].reshape(1, H))


# speedup vs baseline: 3.5656x; 1.0347x over previous
import functools

import jax
import jax.numpy as jnp
from jax import lax
from jax.experimental import pallas as pl
from jax.experimental.pallas import tpu as pltpu
from jax.experimental.pallas import tpu_sc as plsc

N = 10000
E = 320000
H = 128
RBF = 16
G = 64

NC = 2
NS = 16
NW = NC * NS
EPW = E // NW
CH = 80
NCHUNK = EPW // CH
NPAD = 10240
CHP = 400
ZROWS = 80

NB = 1000
EPAD = 327680
EB = 4096
GRID_N = N // NB
GRID_E = EPAD // EB

_mesh = plsc.VectorSubcoreMesh(
    core_axis_name="c", subcore_axis_name="s", num_cores=NC, num_subcores=NS)

_f32 = jnp.float32


def _zero16():
    return jnp.zeros((16,), _f32)


def _iota16():
    return lax.iota(jnp.int32, 16)


@functools.partial(
    pl.kernel,
    mesh=_mesh,
    out_type=[jax.ShapeDtypeStruct((EPAD, H), _f32),
              jax.ShapeDtypeStruct((EPAD,), _f32)],
    scratch_types=(
        [pltpu.VMEM((N * 4,), _f32)]
        + 2 * [pltpu.VMEM((CH,), jnp.int32),
               pltpu.VMEM((CH,), jnp.int32),
               pltpu.VMEM((CH, H), _f32),
               pltpu.VMEM((CH, H), _f32),
               pltpu.VMEM((CH,), _f32),
               pltpu.SemaphoreType.DMA,
               pltpu.SemaphoreType.DMA,
               pltpu.SemaphoreType.DMA]
    ),
    compiler_params=pltpu.CompilerParams(needs_layout_passes=False),
)
def _sc_gather(ah_hbm, bh_hbm, posf_hbm, row_hbm, col_hbm,
               s_hbm, dijf_hbm, posv, *bufs):
    wid = lax.axis_index("s") * NC + lax.axis_index("c")
    base = wid * EPW
    pltpu.sync_copy(posf_hbm, posv)
    sets = (bufs[0:8], bufs[8:16])

    def fire_idx(c, st):
        rowch, colch, isem = st[0], st[1], st[5]
        off = base + c * CH
        pltpu.async_copy(row_hbm.at[pl.ds(off, CH)], rowch, isem)
        pltpu.async_copy(col_hbm.at[pl.ds(off, CH)], colch, isem)

    def wait_idx(st):
        rowch, colch, isem = st[0], st[1], st[5]
        pltpu.make_async_copy(row_hbm.at[pl.ds(0, CH)], rowch, isem).wait()
        pltpu.make_async_copy(col_hbm.at[pl.ds(0, CH)], colch, isem).wait()

    def fire_gath(st):
        rowch, colch, bufa, bufb, gsem = st[0], st[1], st[2], st[3], st[6]
        pltpu.async_copy(ah_hbm.at[rowch], bufa, gsem)
        pltpu.async_copy(bh_hbm.at[colch], bufb, gsem)

    def wait_gath(st):
        bufa, bufb, gsem = st[2], st[3], st[6]
        pltpu.make_async_copy(ah_hbm.at[pl.ds(0, CH)], bufa, gsem).wait()
        pltpu.make_async_copy(bh_hbm.at[pl.ds(0, CH)], bufb, gsem).wait()

    def fire_wb(c, st):
        bufa, dbuf, wsem = st[2], st[4], st[7]
        off = base + c * CH
        pltpu.async_copy(bufa, s_hbm.at[pl.ds(off, CH)], wsem)
        pltpu.async_copy(dbuf, dijf_hbm.at[pl.ds(off, CH)], wsem)

    def wait_wb(st):
        bufa, dbuf, wsem = st[2], st[4], st[7]
        pltpu.make_async_copy(bufa, s_hbm.at[pl.ds(0, CH)], wsem).wait()
        pltpu.make_async_copy(dbuf, dijf_hbm.at[pl.ds(0, CH)], wsem).wait()

    def compute(st):
        rowch, colch, bufa, bufb, dbuf = st[0], st[1], st[2], st[3], st[4]

        def addrow(r, _):
            for l8 in range(H // 16):
                sl = pl.ds(l8 * 16, 16)
                bufa[r, sl] = bufa[r, sl] + bufb[r, sl]
            return 0

        lax.fori_loop(0, CH, addrow, 0)
        for g in range(CH // 16):
            rv4 = rowch[pl.ds(g * 16, 16)] * 4
            cv4 = colch[pl.ds(g * 16, 16)] * 4
            d0 = plsc.load_gather(posv, [rv4]) - plsc.load_gather(posv, [cv4])
            d1 = (plsc.load_gather(posv, [rv4 + 1])
                  - plsc.load_gather(posv, [cv4 + 1]))
            d2 = (plsc.load_gather(posv, [rv4 + 2])
                  - plsc.load_gather(posv, [cv4 + 2]))
            dbuf[pl.ds(g * 16, 16)] = d0 * d0 + d1 * d1 + d2 * d2

    def step(c, p):
        sp, sq = sets[p], sets[1 - p]
        wait_gath(sp)
        wait_idx(sq)

        @pl.when(c >= 1)
        def _():
            wait_wb(sq)

        fire_gath(sq)
        compute(sp)

        @pl.when(c + 2 <= NCHUNK - 1)
        def _():
            fire_idx(c + 2, sp)

        fire_wb(c, sp)

    fire_idx(0, sets[0])
    fire_idx(1, sets[1])
    wait_idx(sets[0])
    fire_gath(sets[0])

    def pair(j, _):
        step(2 * j, 0)
        step(2 * j + 1, 1)
        return 0

    lax.fori_loop(0, (NCHUNK - 1) // 2, pair, 0)
    c_last = NCHUNK - 1
    sp, sq = sets[0], sets[1]
    wait_gath(sp)
    wait_wb(sq)
    compute(sp)
    fire_wb(c_last, sp)
    wait_wb(sp)


@functools.partial(
    pl.kernel,
    mesh=_mesh,
    out_type=jax.ShapeDtypeStruct((NC, NPAD, H), _f32),
    scratch_types=(
        [pltpu.VMEM((ZROWS, H), _f32),
         pltpu.VMEM_SHARED((NPAD, H), _f32)]
        + [pltpu.VMEM((CH,), jnp.int32),
           pltpu.VMEM((CH, H), _f32),
           pltpu.SemaphoreType.DMA]
        + [pltpu.VMEM((CH,), jnp.int32),
           pltpu.VMEM((CH, H), _f32),
           pltpu.SemaphoreType.DMA]
    ),
    compiler_params=pltpu.CompilerParams(needs_layout_passes=False),
)
def _sc_scatter(p_hbm, row_hbm, q_hbm, zbuf, acc, *bufs):
    cid = lax.axis_index("c")
    sid = lax.axis_index("s")
    wid = sid * NC + cid
    base = wid * EPW
    rbase = sid * (NPAD // NS)
    sets = (bufs[0:3], bufs[3:6])

    def fire(c, st):
        rowch, bufp, sem = st
        off = base + c * CH
        pltpu.async_copy(row_hbm.at[pl.ds(off, CH)], rowch, sem)
        pltpu.async_copy(p_hbm.at[pl.ds(off, CH)], bufp, sem)

    def wait(st):
        rowch, bufp, sem = st
        pltpu.make_async_copy(row_hbm.at[pl.ds(0, CH)], rowch, sem).wait()
        pltpu.make_async_copy(p_hbm.at[pl.ds(0, CH)], bufp, sem).wait()

    fire(0, sets[0])
    fire(1, sets[1])

    def zrow(r, _):
        for l8 in range(H // 16):
            zbuf[r, pl.ds(l8 * 16, 16)] = _zero16()
        return 0

    lax.fori_loop(0, ZROWS, zrow, 0)
    for t in range(NPAD // NS // ZROWS):
        pltpu.sync_copy(zbuf, acc.at[pl.ds(rbase + t * ZROWS, ZROWS)])
    plsc.subcore_barrier()

    def step(c, p):
        st = sets[p]
        wait(st)
        pltpu.sync_copy(st[1], acc.at[st[0]], add=True)

        @pl.when(c + 2 <= NCHUNK - 1)
        def _():
            fire(c + 2, st)

    def pair(j, _):
        step(2 * j, 0)
        step(2 * j + 1, 1)
        return 0

    lax.fori_loop(0, (NCHUNK - 1) // 2, pair, 0)
    step(NCHUNK - 1, 0)
    plsc.subcore_barrier()
    for t in range(NPAD // NS // ZROWS):
        pltpu.sync_copy(acc.at[pl.ds(rbase + t * ZROWS, ZROWS)],
                        q_hbm.at[cid, pl.ds(rbase + t * ZROWS, ZROWS)])


@functools.partial(
    pl.kernel,
    mesh=_mesh,
    out_type=jax.ShapeDtypeStruct((NW * NPAD * 4,), _f32),
    scratch_types=[
        pltpu.VMEM((N * 4,), _f32),
        pltpu.VMEM((CHP,), jnp.int32),
        pltpu.VMEM((CHP,), jnp.int32),
        pltpu.VMEM((CHP,), _f32),
        pltpu.VMEM((NPAD * 4,), _f32),
    ],
    compiler_params=pltpu.CompilerParams(use_tc_tiling_on_sc=False,
                                         needs_layout_passes=False),
)
def _sc_scatter_pos(sf_hbm, row_hbm, col_hbm, posf_hbm, o_hbm,
                    posv, rowch, colch, sch, acc2):
    wid = lax.axis_index("s") * NC + lax.axis_index("c")
    base = wid * EPW
    pltpu.sync_copy(posf_hbm, posv)

    def zstep(j, _):
        acc2[pl.ds(j * 16, 16)] = _zero16()
        return 0

    lax.fori_loop(0, NPAD * 4 // 16, zstep, 0)

    def step(k, _):
        off = base + k * CHP
        pltpu.sync_copy(row_hbm.at[pl.ds(off, CHP)], rowch)
        pltpu.sync_copy(col_hbm.at[pl.ds(off, CHP)], colch)
        pltpu.sync_copy(sf_hbm.at[pl.ds(off, CHP)], sch)
        for g in range(CHP // 16):
            rv4 = rowch[pl.ds(g * 16, 16)] * 4
            cv4 = colch[pl.ds(g * 16, 16)] * 4
            sv = sch[pl.ds(g * 16, 16)]
            for c3 in range(3):
                xr = plsc.load_gather(posv, [rv4 + c3])
                xc = plsc.load_gather(posv, [cv4 + c3])
                plsc.addupdate_scatter(acc2, [rv4 + c3], (xr - xc) * sv)
        return 0

    lax.fori_loop(0, EPW // CHP, step, 0)
    pltpu.sync_copy(acc2, o_hbm.at[pl.ds(wid * (NPAD * 4), NPAD * 4)])


def _full(shape):
    return pl.BlockSpec(shape, lambda i: (0,) * len(shape))


def _silu(v):
    return v * jax.nn.sigmoid(v)


def _build_body(x_ref, emb_ref, wr_ref, wc_ref, h_ref, a_ref, b_ref):
    ids = lax.broadcasted_iota(jnp.int32, (NB, 100), 1)
    oh = (x_ref[...] == ids).astype(_f32)
    h = jnp.dot(oh, emb_ref[...], preferred_element_type=_f32)
    h_ref[...] = h
    a_ref[...] = jnp.dot(h, wr_ref[...], preferred_element_type=_f32)
    b_ref[...] = jnp.dot(h, wc_ref[...], preferred_element_type=_f32)


_build_tables = pl.pallas_call(
    _build_body,
    grid=(GRID_N,),
    in_specs=[
        pl.BlockSpec((NB, 1), lambda i: (i, 0)),
        _full((100, H)),
        _full((H, H)),
        _full((H, H)),
    ],
    out_specs=[
        pl.BlockSpec((NB, H), lambda i: (i, 0)),
        pl.BlockSpec((NB, H), lambda i: (i, 0)),
        pl.BlockSpec((NB, H), lambda i: (i, 0)),
    ],
    out_shape=[
        jax.ShapeDtypeStruct((N, H), _f32),
        jax.ShapeDtypeStruct((N, H), _f32),
        jax.ShapeDtypeStruct((N, H), _f32),
    ],
)


DB = EB // 128


def _edge_body(s_ref, dij_ref, ea_ref, wa_ref, wd_ref, b1_ref,
               ew2_ref, b2_ref, cwr_ref, cb_ref, m_ref, sp_ref):
    r_of_e = lax.broadcasted_iota(jnp.int32, (EB, DB), 0) // 128
    sel_r = (r_of_e == lax.broadcasted_iota(jnp.int32, (EB, DB), 1)).astype(_f32)
    l_of_e = lax.broadcasted_iota(jnp.int32, (EB, 128), 0) % 128
    sel_l = (l_of_e == lax.broadcasted_iota(jnp.int32, (EB, 128), 1)).astype(_f32)
    dij = jnp.sum(jnp.dot(sel_r, dij_ref[...], preferred_element_type=_f32)
                  * sel_l, axis=-1, keepdims=True)
    pre = (s_ref[...]
           + jnp.dot(ea_ref[...], wa_ref[...], preferred_element_type=_f32)
           + dij * wd_ref[...] + b1_ref[...])
    m1 = _silu(pre)
    m = _silu(jnp.dot(m1, ew2_ref[...],
                      preferred_element_type=_f32) + b2_ref[...])
    w = _silu(jnp.sum(m * cwr_ref[...], axis=-1, keepdims=True) + cb_ref[...])
    sval = w / (jnp.sqrt(dij) + 1e-8)
    m_ref[...] = m
    sp_ref[...] = lax.dot_general(sel_r, sel_l * sval, (((0,), (0,)), ((), ())),
                                  preferred_element_type=_f32)


_edge_mlp = pl.pallas_call(
    _edge_body,
    grid=(GRID_E,),
    in_specs=[
        pl.BlockSpec((EB, H), lambda i: (i, 0)),
        pl.BlockSpec((DB, 128), lambda i: (i, 0)),
        pl.BlockSpec((EB, RBF), lambda i: (i, 0)),
        _full((RBF, H)),
        _full((1, H)),
        _full((1, H)),
        _full((H, H)),
        _full((1, H)),
        _full((1, H)),
        _full((1, 1)),
    ],
    out_specs=[
        pl.BlockSpec((EB, H), lambda i: (i, 0)),
        pl.BlockSpec((DB, 128), lambda i: (i, 0)),
    ],
    out_shape=[
        jax.ShapeDtypeStruct((EPAD, H), _f32),
        jax.ShapeDtypeStruct((EPAD // 128, 128), _f32),
    ],
)


def _node_body(h_ref, pos_ref, q_ref, q2_ref, w1a_ref, w1b_ref, b1_ref,
               w2_ref, b2_ref, wr_ref, wc_ref,
               h_out, pos_out, a_ref, b_ref):
    h = h_ref[...]
    agg = q_ref[0] + q_ref[1]
    dpos = jnp.sum(q2_ref[...], axis=0)
    nh = _silu(jnp.dot(h, w1a_ref[...], preferred_element_type=_f32)
               + jnp.dot(agg, w1b_ref[...], preferred_element_type=_f32)
               + b1_ref[...])
    hn = h + jnp.dot(nh, w2_ref[...], preferred_element_type=_f32) + b2_ref[...]
    posn = pos_ref[...] + dpos
    h_out[...] = hn
    pos_out[...] = posn
    a_ref[...] = jnp.dot(hn, wr_ref[...], preferred_element_type=_f32)
    b_ref[...] = jnp.dot(hn, wc_ref[...], preferred_element_type=_f32)


_node_update = pl.pallas_call(
    _node_body,
    grid=(GRID_N,),
    in_specs=[
        pl.BlockSpec((NB, H), lambda i: (i, 0)),
        pl.BlockSpec((NB, 4), lambda i: (i, 0)),
        pl.BlockSpec((NC, NB, H), lambda i: (0, i, 0)),
        pl.BlockSpec((NW, NB, 4), lambda i: (0, i, 0)),
        _full((H, H)),
        _full((H, H)),
        _full((1, H)),
        _full((H, H)),
        _full((1, H)),
        _full((H, H)),
        _full((H, H)),
    ],
    out_specs=[
        pl.BlockSpec((NB, H), lambda i: (i, 0)),
        pl.BlockSpec((NB, 4), lambda i: (i, 0)),
        pl.BlockSpec((NB, H), lambda i: (i, 0)),
        pl.BlockSpec((NB, H), lambda i: (i, 0)),
    ],
    out_shape=[
        jax.ShapeDtypeStruct((N, H), _f32),
        jax.ShapeDtypeStruct((N, 4), _f32),
        jax.ShapeDtypeStruct((N, H), _f32),
        jax.ShapeDtypeStruct((N, H), _f32),
    ],
)


def _node_last_body(h_ref, q_ref, w1a_ref, w1b_ref, b1_ref,
                    w2_ref, b2_ref, h_out):
    h = h_ref[...]
    agg = q_ref[0] + q_ref[1]
    nh = _silu(jnp.dot(h, w1a_ref[...], preferred_element_type=_f32)
               + jnp.dot(agg, w1b_ref[...], preferred_element_type=_f32)
               + b1_ref[...])
    h_out[...] = h + jnp.dot(nh, w2_ref[...],
                             preferred_element_type=_f32) + b2_ref[...]


_node_update_last = pl.pallas_call(
    _node_last_body,
    grid=(GRID_N,),
    in_specs=[
        pl.BlockSpec((NB, H), lambda i: (i, 0)),
        pl.BlockSpec((NC, NB, H), lambda i: (0, i, 0)),
        _full((H, H)),
        _full((H, H)),
        _full((1, H)),
        _full((H, H)),
        _full((1, H)),
    ],
    out_specs=pl.BlockSpec((NB, H), lambda i: (i, 0)),
    out_shape=jax.ShapeDtypeStruct((N, H), _f32),
)


def _final_body(h_ref, b_ref, lw_ref, lb_ref, out_ref, acc, cnt):
    i = pl.program_id(0)

    @pl.when(i == 0)
    def _():
        acc[...] = jnp.zeros_like(acc)
        cnt[...] = jnp.zeros_like(cnt)

    ids = lax.broadcasted_iota(jnp.int32, (NB, G), 1)
    oh = (b_ref[...] == ids).astype(_f32)
    acc[...] += lax.dot_general(oh, h_ref[...], (((0,), (0,)), ((), ())),
                                preferred_element_type=_f32)
    cnt[...] += jnp.sum(oh, axis=0, keepdims=True)

    @pl.when(i == GRID_N - 1)
    def _():
        mean = acc[...] / jnp.maximum(cnt[...].reshape(G, 1), 1.0)
        out_ref[...] = jnp.dot(jnp.maximum(mean, 0.0), lw_ref[...],
                               preferred_element_type=_f32) + lb_ref[...]


_final_pool = pl.pallas_call(
    _final_body,
    grid=(GRID_N,),
    in_specs=[
        pl.BlockSpec((NB, H), lambda i: (i, 0)),
        pl.BlockSpec((NB, 1), lambda i: (i, 0)),
        _full((H, H)),
        _full((1, H)),
    ],
    out_specs=pl.BlockSpec((G, H), lambda i: (0, 0)),
    out_shape=jax.ShapeDtypeStruct((G, H), _f32),
    scratch_shapes=[
        pltpu.VMEM((G, H), _f32),
        pltpu.VMEM((1, G), _f32),
    ],
    compiler_params=pltpu.CompilerParams(
        dimension_semantics=("arbitrary",)),
)


def kernel(x, pos, edge_index, edge_attr, batch, params):
    row = edge_index[0].astype(jnp.int32)
    col = edge_index[1].astype(jnp.int32)
    x2 = x.astype(jnp.int32).reshape(N, 1)
    batch2 = batch.astype(jnp.int32).reshape(N, 1)
    pos4 = jnp.pad(pos.astype(_f32), ((0, 0), (0, 1)))
    ea_pad = jnp.pad(edge_attr.astype(_f32), ((0, EPAD - E), (0, 0)))

    layers = params["layers"]
    sliced = []
    for p in layers:
        sliced.append(dict(
            wr=p["e_w1"][:H],
            wc=p["e_w1"][H:2 * H],
            wd=p["e_w1"][2 * H:2 * H + 1],
            wa=p["e_w1"][2 * H + 1:],
            b1=p["e_b1"].reshape(1, H),
            ew2=p["e_w2"],
            b2=p["e_b2"].reshape(1, H),
            cwr=p["c_w"].reshape(1, H),
            cb=p["c_b"].reshape(1, 1),
            w1a=p["n_w1"][:H],
            w1b=p["n_w1"][H:],
            nb1=p["n_b1"].reshape(1, H),
            w2=p["n_w2"],
            nb2=p["n_b2"].reshape(1, H),
        ))

    h, ah, bh = _build_tables(x2, params["emb"],
                              sliced[0]["wr"], sliced[0]["wc"])
    for li, s in enumerate(sliced):
        posf = pos4.reshape(N * 4)
        ssum, dijf = _sc_gather(ah, bh, posf, row, col)
        dijp = dijf.reshape(EPAD // 128, 128)
        m, sp = _edge_mlp(ssum, dijp, ea_pad, s["wa"], s["wd"],
                          s["b1"], s["ew2"], s["b2"], s["cwr"], s["cb"])
        q = _sc_scatter(m, row)
        if li + 1 < len(sliced):
            sf = sp.reshape(EPAD)
            q2f = _sc_scatter_pos(sf, row, col, posf)
            q2 = q2f.reshape(NW, NPAD, 4)[:, :N]
            nxt = sliced[li + 1]
            h, pos4, ah, bh = _node_update(h, pos4, q, q2,
                                           s["w1a"], s["w1b"], s["nb1"],
                                           s["w2"], s["nb2"],
                                           nxt["wr"], nxt["wc"])
        else:
            h = _node_update_last(h, q, s["w1a"], s["w1b"], s["nb1"],
                                  s["w2"], s["nb2"])

    return _final_pool(h, batch2, params["lin_w"],
                       params["lin_b"].reshape(1, H))
